# Initial kernel scaffold; baseline (speedup 1.0000x reference)
import jax
import jax.numpy as jnp
from jax.experimental import pallas as pl


def kernel(x, edge_index, edge_attr, batch, node_emb_W, edge_emb_W, edge_enc_W, edge_enc_b, pre_W, pre_b, post_W, post_b, lin_W, lin_b, bn_gamma, bn_beta, mlp_W1, mlp_b1, mlp_W2, mlp_b2, mlp_W3, mlp_b3):
    raise NotImplementedError("write your pallas kernel here")


# speedup vs baseline: 22.8489x; 22.8489x over previous
import functools
import math

import jax
import jax.numpy as jnp
from jax import lax
from jax.experimental import pallas as pl
from jax.experimental.pallas import tpu as pltpu
from jax.experimental.pallas import tpu_sc as plsc

N = 10000
E = 160000
G = 512
NL = 4
TW = 5
F = 375
FP = 384
AVG_LOG = math.log(17.0)
SUB = 16
NSUB = N // SUB
KB = 64
NW = 32
EPAD = E + KB + 8
BN = 1000
SQ5 = math.sqrt(1e-5)

f32 = jnp.float32
i32 = jnp.int32


def _emb_body(x_ref, w_ref, o_ref):
    oh = (x_ref[...] == lax.broadcasted_iota(i32, (BN, 21), 1)).astype(f32)
    o_ref[...] = jnp.dot(oh, w_ref[...], preferred_element_type=f32)


def _ttab_body(ew_ref, encw_ref, encb_ref, wep_ref, o_ref):
    for l in range(NL):
        etab = jnp.dot(ew_ref[...], encw_ref[l], preferred_element_type=f32) + encb_ref[l]
        o_ref[l] = jnp.dot(etab, wep_ref[l], preferred_element_type=f32)


def _prep_body(h_ref, wd_ref, ws_ref, pb_ref, a_ref, b_ref):
    h = h_ref[...]
    a_ref[...] = jnp.dot(h, wd_ref[...], preferred_element_type=f32) + pb_ref[...]
    b_ref[...] = jnp.dot(h, ws_ref[...], preferred_element_type=f32)


def _combine_body(h_ref, a_ref, s_ref, ptr2_ref, bd_ref, p0_ref, b75_ref, y_ref):
    p2 = ptr2_ref[0]
    cnt = (p2[:, 1] - p2[:, 0]).astype(f32)[:, None]
    has = cnt > 0.0
    cc = jnp.maximum(cnt, 1.0)
    inv = 1.0 / cc
    a = a_ref[...][:, :F]
    s1 = s_ref[0][:, :F]
    s2 = s_ref[1][:, :F]
    mnu = s_ref[2][:, :F]
    mxu = s_ref[3][:, :F]
    m1 = s1 * inv
    mean = jnp.where(has, a + m1, 0.0)
    var = jax.nn.relu(s2 * inv - m1 * m1)
    std = jnp.where(has, jnp.sqrt(var + 1e-5), SQ5)
    mn = jnp.where(has, a + mnu, 0.0)
    mx = jnp.where(has, a + mxu, 0.0)
    lg = jnp.log(cc + 1.0)
    amp = lg * (1.0 / AVG_LOG)
    att = AVG_LOG / lg
    y = jnp.dot(h_ref[...], p0_ref[...], preferred_element_type=f32) + b75_ref[...]
    for ai, m in enumerate((mean, mn, mx, std)):
        y = y + jnp.dot(m, bd_ref[ai, 0], preferred_element_type=f32)
        y = y + amp * jnp.dot(m, bd_ref[ai, 1], preferred_element_type=f32)
        y = y + att * jnp.dot(m, bd_ref[ai, 2], preferred_element_type=f32)
    y_ref[...] = y


def _bn_body(y_ref, g_ref, b_ref, o_ref):
    y = y_ref[...]
    mu = jnp.mean(y, axis=0, keepdims=True)
    var = jnp.mean((y - mu) ** 2, axis=0, keepdims=True)
    o_ref[...] = jax.nn.relu((y - mu) * lax.rsqrt(var + 1e-5) * g_ref[...] + b_ref[...])


def _pool_body(h_ref, bat_ref, w1_ref, b1_ref, w2_ref, b2_ref, w3_ref, b3_ref, o_ref):
    oht = (bat_ref[...] == lax.broadcasted_iota(i32, (G, N), 0)).astype(f32)
    g = jnp.dot(oht, h_ref[...], preferred_element_type=f32)
    g = jax.nn.relu(jnp.dot(g, w1_ref[...], preferred_element_type=f32) + b1_ref[...])
    g = jax.nn.relu(jnp.dot(g, w2_ref[...], preferred_element_type=f32) + b2_ref[...])
    o_ref[...] = jnp.dot(g, w3_ref[...], preferred_element_type=f32) + b3_ref[...]


def _sc_edges(b_hbm, srcs_hbm, dsts_hbm, attrs_hbm, sptr_hbm, ttab_hbm, out_hbm,
              sptr_v, ttab_v, idx_v, d_v, a_v, gbuf, acc, sem):
    cid = lax.axis_index("c")
    sid = lax.axis_index("s")
    wid = sid * 2 + cid
    pltpu.sync_copy(sptr_hbm, sptr_v)
    pltpu.sync_copy(ttab_hbm, ttab_v)

    def process(r):
        pvec = sptr_v[pl.ds(r, 16)]
        e0 = pvec[0]
        e1 = pvec[1]

        def initrow(j, _):
            for c in range(FP // 16):
                sl = pl.ds(c * 16, 16)
                acc[0, j, sl] = jnp.zeros((16,), f32)
                acc[1, j, sl] = jnp.zeros((16,), f32)
                acc[2, j, sl] = jnp.full((16,), 3e38, f32)
                acc[3, j, sl] = jnp.full((16,), -3e38, f32)
            return 0

        lax.fori_loop(0, SUB, initrow, 0)

        a0 = jnp.bitwise_and(e0, jnp.int32(-8))
        nb = (e1 - a0 + (KB - 1)) // KB

        def batch_body(b, _):
            base = pl.multiple_of(a0 + b * KB, 8)
            pltpu.sync_copy(srcs_hbm.at[pl.ds(base, KB)], idx_v)
            pltpu.sync_copy(dsts_hbm.at[pl.ds(base, KB)], d_v.at[pl.ds(0, KB)])
            pltpu.sync_copy(attrs_hbm.at[pl.ds(base, KB)], a_v.at[pl.ds(0, KB)])
            pltpu.async_copy(b_hbm.at[idx_v], gbuf, sem).wait()

            def edge_body(j, _):
                gidx = base + j

                @pl.when((gidx >= e0) & (gidx < e1))
                def _():
                    dl = d_v[pl.ds(j, 16)][0] - r * SUB
                    at = a_v[pl.ds(j, 16)][0]
                    for c in range(FP // 16):
                        sl = pl.ds(c * 16, 16)
                        u = gbuf[j, sl] + ttab_v[at, sl]
                        acc[0, dl, sl] = acc[0, dl, sl] + u
                        acc[1, dl, sl] = acc[1, dl, sl] + u * u
                        acc[2, dl, sl] = jnp.minimum(acc[2, dl, sl], u)
                        acc[3, dl, sl] = jnp.maximum(acc[3, dl, sl], u)

                return 0

            lax.fori_loop(0, KB, edge_body, 0)
            return 0

        lax.fori_loop(0, nb, batch_body, 0)
        for st in range(4):
            pltpu.sync_copy(acc.at[st], out_hbm.at[st, pl.ds(r * SUB, SUB)])

    def outer(i, _):
        r = wid + i * NW

        @pl.when(r < NSUB)
        def _():
            process(r)

        return 0

    lax.fori_loop(0, (NSUB + NW - 1) // NW, outer, 0)


def _make_sc_call():
    mesh = plsc.VectorSubcoreMesh(core_axis_name="c", subcore_axis_name="s")
    return functools.partial(
        pl.kernel,
        mesh=mesh,
        out_type=jax.ShapeDtypeStruct((4, N, FP), f32),
        scratch_types=[
            pltpu.VMEM((NSUB + 23, ), i32),
            pltpu.VMEM((4, FP), f32),
            pltpu.VMEM((KB,), i32),
            pltpu.VMEM((KB + 16,), i32),
            pltpu.VMEM((KB + 16,), i32),
            pltpu.VMEM((KB, FP), f32),
            pltpu.VMEM((4, SUB, FP), f32),
            pltpu.SemaphoreType.DMA,
        ],
    )(_sc_edges)


def _tc_call(body, grid, in_specs, out_specs, out_shape):
    return pl.pallas_call(body, grid=grid, in_specs=in_specs,
                          out_specs=out_specs, out_shape=out_shape)


def kernel(x, edge_index, edge_attr, batch,
           node_emb_W, edge_emb_W, edge_enc_W, edge_enc_b,
           pre_W, pre_b, post_W, post_b, lin_W, lin_b,
           bn_gamma, bn_beta,
           mlp_W1, mlp_b1, mlp_W2, mlp_b2, mlp_W3, mlp_b3):
    src = edge_index[0].astype(i32)
    dst = edge_index[1].astype(i32)
    perm = jnp.argsort(dst)
    src_s = src[perm]
    dst_s = dst[perm]
    attr_s = edge_attr[perm].astype(i32)
    ptr = jnp.searchsorted(dst_s, jnp.arange(N + 1, dtype=i32)).astype(i32)
    sptr = jnp.pad(ptr[::SUB], (0, 22))
    srcs_p = jnp.pad(src_s, (0, EPAD - E))
    dsts_p = jnp.pad(dst_s, (0, EPAD - E))
    attrs_p = jnp.pad(attr_s, (0, EPAD - E))
    ptr2 = jnp.stack([ptr[:-1], ptr[1:]], axis=-1).reshape(N // BN, BN, 2)

    preWf = jnp.transpose(pre_W, (0, 2, 1, 3)).reshape(NL, 225, F)
    padF = [(0, 0), (0, 0), (0, FP - F)]
    Wd_p = jnp.pad(preWf[:, 0:75, :], padF)
    Ws_p = jnp.pad(preWf[:, 75:150, :], padF)
    We_p = jnp.pad(preWf[:, 150:225, :], padF)
    preb_p = jnp.pad(pre_b.reshape(NL, F), [(0, 0), (0, FP - F)])[:, None, :]

    P0 = jnp.transpose(post_W[:, :, 0:75, :], (0, 2, 1, 3)).reshape(NL, 75, 75)
    P0l = jnp.einsum('lfp,lpq->lfq', P0, lin_W)
    BD = jnp.zeros((NL, 4, 3, F, 75), f32)
    for s in range(3):
        for a in range(4):
            blk = post_W[:, :, 75 + s * 300 + a * 75: 75 + s * 300 + (a + 1) * 75, :]
            for t in range(TW):
                BD = BD.at[:, a, s, t * 75:(t + 1) * 75, t * 15:(t + 1) * 15].set(blk[:, t])
    BDl = jnp.einsum('lasfp,lpq->lasfq', BD, lin_W)
    b75 = jnp.einsum('lp,lpq->lq', post_b.reshape(NL, 75), lin_W) + lin_b

    h = _tc_call(
        _emb_body, (N // BN,),
        [pl.BlockSpec((BN, 1), lambda i: (i, 0)),
         pl.BlockSpec((21, 75), lambda i: (0, 0))],
        pl.BlockSpec((BN, 75), lambda i: (i, 0)),
        jax.ShapeDtypeStruct((N, 75), f32),
    )(x.astype(i32), node_emb_W)

    ttab = _tc_call(
        _ttab_body, (1,),
        [pl.BlockSpec((4, 50), lambda i: (0, 0)),
         pl.BlockSpec((NL, 50, 75), lambda i: (0, 0, 0)),
         pl.BlockSpec((NL, 1, 75), lambda i: (0, 0, 0)),
         pl.BlockSpec((NL, 75, FP), lambda i: (0, 0, 0))],
        pl.BlockSpec((NL, 4, FP), lambda i: (0, 0, 0)),
        jax.ShapeDtypeStruct((NL, 4, FP), f32),
    )(edge_emb_W, edge_enc_W, edge_enc_b[:, None, :], We_p)

    sc_call = _make_sc_call()

    for l in range(NL):
        A, B = _tc_call(
            _prep_body, (N // BN,),
            [pl.BlockSpec((BN, 75), lambda i: (i, 0)),
             pl.BlockSpec((75, FP), lambda i: (0, 0)),
             pl.BlockSpec((75, FP), lambda i: (0, 0)),
             pl.BlockSpec((1, FP), lambda i: (0, 0))],
            [pl.BlockSpec((BN, FP), lambda i: (i, 0)),
             pl.BlockSpec((BN, FP), lambda i: (i, 0))],
            [jax.ShapeDtypeStruct((N, FP), f32),
             jax.ShapeDtypeStruct((N, FP), f32)],
        )(h, Wd_p[l], Ws_p[l], preb_p[l])

        stats = sc_call(B, srcs_p, dsts_p, attrs_p, sptr, ttab[l])

        y = _tc_call(
            _combine_body, (N // BN,),
            [pl.BlockSpec((BN, 75), lambda i: (i, 0)),
             pl.BlockSpec((BN, FP), lambda i: (i, 0)),
             pl.BlockSpec((4, BN, FP), lambda i: (0, i, 0)),
             pl.BlockSpec((1, BN, 2), lambda i: (i, 0, 0)),
             pl.BlockSpec((4, 3, F, 75), lambda i: (0, 0, 0, 0)),
             pl.BlockSpec((75, 75), lambda i: (0, 0)),
             pl.BlockSpec((1, 75), lambda i: (0, 0))],
            pl.BlockSpec((BN, 75), lambda i: (i, 0)),
            jax.ShapeDtypeStruct((N, 75), f32),
        )(h, A, stats, ptr2, BDl[l], P0l[l], b75[l][None, :])

        h = _tc_call(
            _bn_body, (1,),
            [pl.BlockSpec((N, 75), lambda i: (0, 0)),
             pl.BlockSpec((1, 75), lambda i: (0, 0)),
             pl.BlockSpec((1, 75), lambda i: (0, 0))],
            pl.BlockSpec((N, 75), lambda i: (0, 0)),
            jax.ShapeDtypeStruct((N, 75), f32),
        )(y, bn_gamma[l][None, :], bn_beta[l][None, :])

    out = _tc_call(
        _pool_body, (1,),
        [pl.BlockSpec((N, 75), lambda i: (0, 0)),
         pl.BlockSpec((1, N), lambda i: (0, 0)),
         pl.BlockSpec((75, 50), lambda i: (0, 0)),
         pl.BlockSpec((1, 50), lambda i: (0, 0)),
         pl.BlockSpec((50, 25), lambda i: (0, 0)),
         pl.BlockSpec((1, 25), lambda i: (0, 0)),
         pl.BlockSpec((25, 1), lambda i: (0, 0)),
         pl.BlockSpec((1, 1), lambda i: (0, 0))],
        pl.BlockSpec((G, 1), lambda i: (0, 0)),
        jax.ShapeDtypeStruct((G, 1), f32),
    )(h, batch.astype(i32)[None, :], mlp_W1, mlp_b1[None, :],
      mlp_W2, mlp_b2[None, :], mlp_W3, mlp_b3[None, :])
    return out
